# baseline (speedup 1.0000x reference)
import jax
import jax.numpy as jnp
from jax.experimental import pallas as pl
from jax.experimental.pallas import tpu as pltpu

EPS = 1e-8
ROWS_PER_BLOCK = 64


def _taylor_softmax_block(x_ref, o_ref):
    x = x_ref[:]
    m = jnp.max(x, axis=1, keepdims=True)
    t = x - m
    numer = 1.0 + t + jnp.square(t + EPS) * 0.5
    denom = jnp.sum(numer, axis=1, keepdims=True) + EPS
    o_ref[:] = numer * (1.0 / denom)


def kernel(logits):
    n_rows, n_cols = logits.shape
    n_steps = n_rows // ROWS_PER_BLOCK
    mesh = pltpu.create_tensorcore_mesh("core")

    def stateful(refs):
        x_hbm, o_hbm = refs

        @pl.core_map(
            mesh,
            compiler_params=pltpu.CompilerParams(
                vmem_limit_bytes=100 * 1024 * 1024,
            ),
        )
        def _():
            pltpu.emit_pipeline(
                _taylor_softmax_block,
                grid=(n_steps,),
                in_specs=[
                    pl.BlockSpec((ROWS_PER_BLOCK, n_cols), lambda i: (i, 0))
                ],
                out_specs=[
                    pl.BlockSpec((ROWS_PER_BLOCK, n_cols), lambda i: (i, 0))
                ],
                core_axis_name="core",
            )(x_hbm, o_hbm)

    _, out = pl.run_state(stateful)(
        (logits, jnp.zeros(logits.shape, logits.dtype))
    )
    return out


# speedup vs baseline: 1.4922x; 1.4922x over previous
import jax
import jax.numpy as jnp
from jax.experimental import pallas as pl
from jax.experimental.pallas import tpu as pltpu

EPS = 1e-8
ROWS_PER_BLOCK = 64


def _taylor_softmax_block(x_ref, o_ref):
    x = x_ref[:]
    m = jnp.max(x, axis=1, keepdims=True)
    t = x - m
    numer = 1.0 + t + jnp.square(t + EPS) * 0.5
    denom = jnp.sum(numer, axis=1, keepdims=True) + EPS
    o_ref[:] = numer * (1.0 / denom)


def kernel(logits):
    n_rows, n_cols = logits.shape
    grid = (n_rows // ROWS_PER_BLOCK,)
    return pl.pallas_call(
        _taylor_softmax_block,
        grid=grid,
        in_specs=[pl.BlockSpec((ROWS_PER_BLOCK, n_cols), lambda i: (i, 0))],
        out_specs=pl.BlockSpec((ROWS_PER_BLOCK, n_cols), lambda i: (i, 0)),
        out_shape=jax.ShapeDtypeStruct((n_rows, n_cols), logits.dtype),
        compiler_params=pltpu.CompilerParams(
            dimension_semantics=("arbitrary",),
            vmem_limit_bytes=100 * 1024 * 1024,
        ),
    )(logits)
